# Initial kernel scaffold; baseline (speedup 1.0000x reference)
import jax
import jax.numpy as jnp
from jax.experimental import pallas as pl


def kernel(params, drug_x, prot_x, drug_edge_attr, prot_edge_attr, drug_edge_index, prot_edge_index, drug_batch, prot_batch):
    raise NotImplementedError("write your pallas kernel here")


# speedup vs baseline: 9.6376x; 9.6376x over previous
import functools
import math

import jax
import jax.numpy as jnp
import numpy as np
from jax import lax
from jax.experimental import pallas as pl
from jax.experimental.pallas import tpu as pltpu

HID = 128
HEADS = 4
C = HID // HEADS
NG = 64
EPS_BN = 1e-5


def _bn_act(x, stats, g, b):
    m = stats[0:1, :]
    v = stats[1:2, :]
    y = (x - m) / jnp.sqrt(v + EPS_BN) * g + b
    return jnp.where(y > 0, y, 0.2 * y)


def _linear_body(x_ref, w_ref, b_ref, o_ref, *, act):
    x = x_ref[...]
    y = jnp.dot(x, w_ref[...], preferred_element_type=jnp.float32) + b_ref[...]
    if act:
        y = jnp.where(y > 0, y, 0.2 * y)
    o_ref[...] = y


def _linear_bn_body(x_ref, stats_ref, g_ref, b_ref, w_ref, bias_ref, o_ref, *, act):
    x = _bn_act(x_ref[...], stats_ref[...], g_ref[...], b_ref[...])
    y = jnp.dot(x, w_ref[...], preferred_element_type=jnp.float32) + bias_ref[...]
    if act:
        y = jnp.where(y > 0, y, 0.2 * y)
    o_ref[...] = y


def linear_tc(x, w, b, act=False, block=512):
    n, din = x.shape
    dout = w.shape[1]
    nb = pl.cdiv(n, block)
    b2 = b.reshape(1, dout)
    return pl.pallas_call(
        functools.partial(_linear_body, act=act),
        grid=(nb,),
        in_specs=[
            pl.BlockSpec((block, din), lambda i: (i, 0)),
            pl.BlockSpec((din, dout), lambda i: (0, 0)),
            pl.BlockSpec((1, dout), lambda i: (0, 0)),
        ],
        out_specs=pl.BlockSpec((block, dout), lambda i: (i, 0)),
        out_shape=jax.ShapeDtypeStruct((n, dout), jnp.float32),
    )(x, w, b2)


def linear_bn_tc(x, stats, g, b, w, bias, act=False, block=512):
    n, din = x.shape
    dout = w.shape[1]
    nb = pl.cdiv(n, block)
    return pl.pallas_call(
        functools.partial(_linear_bn_body, act=act),
        grid=(nb,),
        in_specs=[
            pl.BlockSpec((block, din), lambda i: (i, 0)),
            pl.BlockSpec((2, din), lambda i: (0, 0)),
            pl.BlockSpec((1, din), lambda i: (0, 0)),
            pl.BlockSpec((1, din), lambda i: (0, 0)),
            pl.BlockSpec((din, dout), lambda i: (0, 0)),
            pl.BlockSpec((1, dout), lambda i: (0, 0)),
        ],
        out_specs=pl.BlockSpec((block, dout), lambda i: (i, 0)),
        out_shape=jax.ShapeDtypeStruct((n, dout), jnp.float32),
    )(x, stats, g.reshape(1, din), b.reshape(1, din), w, bias.reshape(1, dout))


def _tree_sum0(x):
    r = x.shape[0]
    while r > 1:
        r //= 2
        x = x[:r] + x[r:]
    return x


def _stats_body(x_ref, o_ref):
    x = x_ref[...]
    n = x.shape[0]
    m = _tree_sum0(x) / n
    d = x - m
    v = _tree_sum0(d * d) / n
    o_ref[...] = jnp.concatenate([m, v], axis=0)


def stats_tc(x):
    n, d = x.shape
    return pl.pallas_call(
        _stats_body,
        out_shape=jax.ShapeDtypeStruct((2, d), jnp.float32),
    )(x)


def _gat_epilogue_body(parts_ref, res_ref, bias_ref, o_ref):
    p = parts_ref[0] + parts_ref[1]
    num = p[:, :HID]
    dens = [p[:, HID + h:HID + h + 1] for h in range(HEADS)]
    den = jnp.concatenate([jnp.broadcast_to(d, (d.shape[0], C)) for d in dens],
                          axis=1)
    o_ref[...] = num / (den + 1e-16) + res_ref[...] + bias_ref[...]


def gat_epilogue_tc(parts, res, bias, block=512):
    n = res.shape[0]
    nb = pl.cdiv(n, block)
    w = parts.shape[2]
    return pl.pallas_call(
        _gat_epilogue_body,
        grid=(nb,),
        in_specs=[
            pl.BlockSpec((2, block, w), lambda i: (0, i, 0)),
            pl.BlockSpec((block, HID), lambda i: (i, 0)),
            pl.BlockSpec((1, HID), lambda i: (0, 0)),
        ],
        out_specs=pl.BlockSpec((block, HID), lambda i: (i, 0)),
        out_shape=jax.ShapeDtypeStruct((n, HID), jnp.float32),
    )(parts, res, bias.reshape(1, HID))


def _cross_body(q_ref, k_ref, v_ref, qb_ref, kb_ref, o_ref):
    q = q_ref[...]
    k = k_ref[...]
    v = v_ref[...]
    s = jnp.dot(q, k.T, preferred_element_type=jnp.float32) / math.sqrt(HID)
    mask = qb_ref[...].T == kb_ref[...]
    s = jnp.where(mask, s, -jnp.inf)
    mx = jnp.max(s, axis=1, keepdims=True)
    mx = jnp.maximum(mx, -1e30)
    p = jnp.where(mask, jnp.exp(s - mx), 0.0)
    den = jnp.sum(p, axis=1, keepdims=True)
    o = jnp.dot(p, v, preferred_element_type=jnp.float32)
    o_ref[...] = o / jnp.where(den == 0.0, 1.0, den)


def cross_attn_tc(q, k, v, qb, kb, block_q=512):
    nq = q.shape[0]
    nk = k.shape[0]
    nb = pl.cdiv(nq, block_q)
    qb2 = qb.reshape(1, nq).astype(jnp.int32)
    kb2 = kb.reshape(1, nk).astype(jnp.int32)
    return pl.pallas_call(
        _cross_body,
        grid=(nb,),
        in_specs=[
            pl.BlockSpec((block_q, HID), lambda i: (i, 0)),
            pl.BlockSpec((nk, HID), lambda i: (0, 0)),
            pl.BlockSpec((nk, HID), lambda i: (0, 0)),
            pl.BlockSpec((1, block_q), lambda i: (0, i)),
            pl.BlockSpec((1, nk), lambda i: (0, 0)),
        ],
        out_specs=pl.BlockSpec((block_q, HID), lambda i: (i, 0)),
        out_shape=jax.ShapeDtypeStruct((nq, HID), jnp.float32),
    )(q, k, v, qb2, kb2)


def _bn_full(x, g, b):
    n = x.shape[0]
    m = _tree_sum0(x) / n
    d = x - m
    v = _tree_sum0(d * d) / n
    return d / jnp.sqrt(v + EPS_BN) * g + b


def _head_body(gp_ref, gd_ref, l1w_ref, l1b_ref, bn1g_ref, bn1b_ref,
               l2w_ref, l2b_ref, bn2g_ref, bn2b_ref,
               l3w_ref, l3b_ref, bn3g_ref, bn3b_ref,
               l4w_ref, l4b_ref, o_ref):
    x = jnp.concatenate([gp_ref[...], gd_ref[...]], axis=1)
    x = jnp.dot(x, l1w_ref[...], preferred_element_type=jnp.float32) + l1b_ref[...]
    x = _bn_full(x, bn1g_ref[...], bn1b_ref[...])
    x = jnp.where(x > 0, x, 0.2 * x)
    x = jnp.dot(x, l2w_ref[...], preferred_element_type=jnp.float32) + l2b_ref[...]
    x = _bn_full(x, bn2g_ref[...], bn2b_ref[...])
    x = jnp.where(x > 0, x, 0.2 * x)
    x = jnp.dot(x, l3w_ref[...], preferred_element_type=jnp.float32) + l3b_ref[...]
    x = _bn_full(x, bn3g_ref[...], bn3b_ref[...])
    x = jnp.where(x > 0, x, 0.2 * x)
    x = jnp.dot(x, l4w_ref[...], preferred_element_type=jnp.float32) + l4b_ref[...]
    o_ref[...] = x


def head_tc(gp, gd, hp):
    return pl.pallas_call(
        _head_body,
        out_shape=jax.ShapeDtypeStruct((NG, 1), jnp.float32),
    )(gp, gd,
      hp["l1"]["W"], hp["l1"]["b"].reshape(1, -1),
      hp["bn1g"].reshape(1, -1), hp["bn1b"].reshape(1, -1),
      hp["l2"]["W"], hp["l2"]["b"].reshape(1, -1),
      hp["bn2g"].reshape(1, -1), hp["bn2b"].reshape(1, -1),
      hp["l3"]["W"], hp["l3"]["b"].reshape(1, -1),
      hp["bn3g"].reshape(1, -1), hp["bn3b"].reshape(1, -1),
      hp["l4"]["W"], hp["l4"]["b"].reshape(1, -1))


def gat_edge_phase(xl, xr, e, src, dst, att, n):
    msg = xl[src] + xr[dst] + e
    msg = msg.reshape(-1, HEADS, C)
    alpha = (jnp.where(msg > 0, msg, 0.2 * msg) * att[None]).sum(-1)
    alpha = jnp.minimum(alpha, 60.0)
    ex = jnp.exp(alpha)
    num = jax.ops.segment_sum(
        jnp.repeat(ex, C, axis=1) * xl[src], dst, num_segments=n)
    den = jax.ops.segment_sum(ex, dst, num_segments=n)
    part = jnp.concatenate(
        [num, den, jnp.zeros((n, 12), jnp.float32)], axis=1)
    return jnp.stack([part, jnp.zeros_like(part)], axis=0)


def _gat_layer(x, stats, src, dst, ea, lp, n, first):
    wcat = jnp.concatenate(
        [lp["lin_l"]["W"], lp["lin_r"]["W"], lp["res"]["W"]], axis=1)
    bcat = jnp.concatenate(
        [lp["lin_l"]["b"], lp["lin_r"]["b"], jnp.zeros((HID,), jnp.float32)])
    if first:
        z = linear_tc(x, wcat, bcat)
    else:
        z = linear_bn_tc(x, stats, lp["bn_g_prev"], lp["bn_b_prev"], wcat, bcat)
    xl = z[:, :HID]
    xr = z[:, HID:2 * HID]
    res = z[:, 2 * HID:]
    e = linear_tc(ea, lp["lin_e"]["W"], jnp.zeros((HID,), jnp.float32))
    parts = gat_edge_phase(xl, xr, e, src, dst, lp["att"], n)
    y = gat_epilogue_tc(parts, res, lp["bias"])
    return y


def _encode_pallas(x, ei, ea, ep, n):
    src = ei[0].astype(jnp.int32)
    dst = ei[1].astype(jnp.int32)
    layers = ep["layers"]
    y = x
    stats = None
    for i, lp in enumerate(layers):
        lp = dict(lp)
        if i > 0:
            lp["bn_g_prev"] = layers[i - 1]["bn_g"]
            lp["bn_b_prev"] = layers[i - 1]["bn_b"]
        y = _gat_layer(y, stats, src, dst, ea, lp, n, first=(i == 0))
        stats = jnp.stack([jnp.mean(y, axis=0), jnp.var(y, axis=0)], axis=0)
    last = layers[-1]
    return linear_bn_tc(y, stats, last["bn_g"], last["bn_b"],
                        ep["final"]["W"], ep["final"]["b"])


def kernel(params, drug_x, prot_x, drug_edge_attr, prot_edge_attr,
           drug_edge_index, prot_edge_index, drug_batch, prot_batch):
    n_d = drug_x.shape[0]
    n_p = prot_x.shape[0]
    x_d = _encode_pallas(drug_x, drug_edge_index, drug_edge_attr,
                         params["drug_enc"], n_d)
    x_p = _encode_pallas(prot_x, prot_edge_index, prot_edge_attr,
                         params["prot_enc"], n_p)

    ca = params["ca_d2p"]
    q1 = linear_tc(x_p, ca["q"]["W"], ca["q"]["b"])
    k1 = linear_tc(x_d, ca["k"]["W"], ca["k"]["b"])
    v1 = linear_tc(x_d, ca["v"]["W"], ca["v"]["b"])
    a1 = cross_attn_tc(q1, k1, v1, prot_batch, drug_batch)
    a1 = linear_tc(a1, ca["o"]["W"], ca["o"]["b"])

    cb = params["ca_p2d"]
    q2 = linear_tc(x_d, cb["q"]["W"], cb["q"]["b"])
    k2 = linear_tc(x_p, cb["k"]["W"], cb["k"]["b"])
    v2 = linear_tc(x_p, cb["v"]["W"], cb["v"]["b"])
    a2 = cross_attn_tc(q2, k2, v2, drug_batch, prot_batch)
    a2 = linear_tc(a2, cb["o"]["W"], cb["o"]["b"])

    wp = jax.nn.softmax(params["fw_p"], axis=1)[:, 0:2]
    wd = jax.nn.softmax(params["fw_d"], axis=1)[:, 0:2]
    gp = _fuse_pool_col(x_p, a1, wp, prot_batch, params["pool_p"])
    gd = _fuse_pool_col(x_d, a2, wd, drug_batch, params["pool_d"])

    hp = params["head"]
    x0 = jnp.concatenate([gp, gd], axis=1)
    y1 = linear_tc(x0, hp["l1"]["W"], hp["l1"]["b"], block=NG)
    st1 = jnp.stack([jnp.mean(y1, axis=0), jnp.var(y1, axis=0)], axis=0)
    y2 = linear_bn_tc(y1, st1, hp["bn1g"], hp["bn1b"],
                      hp["l2"]["W"], hp["l2"]["b"], block=NG)
    st2 = jnp.stack([jnp.mean(y2, axis=0), jnp.var(y2, axis=0)], axis=0)
    y3 = linear_bn_tc(y2, st2, hp["bn2g"], hp["bn2b"],
                      hp["l3"]["W"], hp["l3"]["b"], block=NG)
    st3 = jnp.stack([jnp.mean(y3, axis=0), jnp.var(y3, axis=0)], axis=0)
    return linear_bn_tc(y3, st3, hp["bn3g"], hp["bn3b"],
                        hp["l4"]["W"], hp["l4"]["b"], block=NG)


def _fuse_gate_body(x_ref, a_ref, w_ref, g1w_ref, g1b_ref, g2w_ref, g2b_ref,
                    xo_ref, g_ref):
    w = w_ref[...]
    x = x_ref[...] * w[0:1, :] + a_ref[...] * w[1:2, :]
    h = jnp.dot(x, g1w_ref[...], preferred_element_type=jnp.float32) + g1b_ref[...]
    h = jnp.where(h > 0, h, 0.2 * h)
    g = jnp.dot(h, g2w_ref[...], preferred_element_type=jnp.float32) + g2b_ref[...]
    xo_ref[...] = x
    g_ref[...] = g


def _fuse_pool_col(x, attn, w, batch, pp, block=1024):
    n = x.shape[0]
    nb = pl.cdiv(n, block)
    xf, g = pl.pallas_call(
        _fuse_gate_body,
        grid=(nb,),
        in_specs=[
            pl.BlockSpec((block, HID), lambda i: (i, 0)),
            pl.BlockSpec((block, HID), lambda i: (i, 0)),
            pl.BlockSpec((2, HID), lambda i: (0, 0)),
            pl.BlockSpec((HID, HID // 2), lambda i: (0, 0)),
            pl.BlockSpec((1, HID // 2), lambda i: (0, 0)),
            pl.BlockSpec((HID // 2, 1), lambda i: (0, 0)),
            pl.BlockSpec((1, 1), lambda i: (0, 0)),
        ],
        out_specs=[
            pl.BlockSpec((block, HID), lambda i: (i, 0)),
            pl.BlockSpec((block, 1), lambda i: (i, 0)),
        ],
        out_shape=[
            jax.ShapeDtypeStruct((n, HID), jnp.float32),
            jax.ShapeDtypeStruct((n, 1), jnp.float32),
        ],
    )(x, attn, w.T, pp["g1"]["W"], pp["g1"]["b"].reshape(1, -1),
      pp["g2"]["W"], pp["g2"]["b"].reshape(1, 1))
    g = g[:, 0]
    gm = jax.ops.segment_max(g, batch, num_segments=NG)
    wgt = jnp.exp(g - gm[batch])
    den = jax.ops.segment_sum(wgt, batch, num_segments=NG)
    wgt = wgt / (den[batch] + 1e-16)
    return jax.ops.segment_sum(wgt[:, None] * xf, batch, num_segments=NG)
